# initial kernel scaffold (unmeasured)
import jax
import jax.numpy as jnp
from jax import lax
from jax.experimental import pallas as pl
from jax.experimental.pallas import tpu as pltpu

N_DEV = 8
ROWS = 512
D_MODEL = 256
D_FF = 512
N_EXP = 16
EXP_PER_DEV = N_EXP // N_DEV
CHUNK = ROWS // N_DEV


def kernel(x, router_W, route_idx, expert_W):
    def body(
        x_ref,
        rw_ref,
        idx_ref,
        ew_ref,
        out_ref,
        rs_buf,
        rs_send_sems,
        rs_recv_sems,
        ag_send_sems,
        ag_recv_sems,
    ):
        my = lax.axis_index("i")
        left = jnp.mod(my - 1, N_DEV)
        right = jnp.mod(my + 1, N_DEV)

        barrier_sem = pltpu.get_barrier_semaphore()
        for nbr in (left, right):
            pltpu.semaphore_signal(
                barrier_sem,
                inc=1,
                device_id=(nbr,),
                device_id_type=pltpu.DeviceIdType.MESH,
            )
        pltpu.semaphore_wait(barrier_sem, 2)

        x_v = x_ref[:, :]
        scores = jnp.dot(x_v, rw_ref[:, :], preferred_element_type=jnp.float32)
        s_max = jnp.max(scores, axis=-1, keepdims=True)
        p = jnp.exp(scores - s_max)
        p = p / jnp.sum(p, axis=-1, keepdims=True)

        idx0 = idx_ref[:, 0:1]
        idx1 = idx_ref[:, 1:2]
        e_iota = lax.broadcasted_iota(jnp.int32, (ROWS, N_EXP), 1)
        g0 = jnp.sum(
            jnp.where(idx0 == e_iota, p, 0.0), axis=-1, keepdims=True
        )
        g1 = jnp.sum(jnp.where(idx1 == e_iota, p, 0.0), axis=-1, keepdims=True)
        gs = g0 + g1
        w0 = g0 / gs
        w1 = g1 / gs

        e_base = my * EXP_PER_DEV
        acc = None
        for k in range(EXP_PER_DEV):
            e = e_base + k
            m = jnp.where(idx0 == e, w0, 0.0) + jnp.where(idx1 == e, w1, 0.0)
            xm = x_v * m
            c = jnp.dot(xm, ew_ref[k, :, :], preferred_element_type=jnp.float32)
            acc = c if acc is None else acc + c
        out_ref[:, :] = acc

        for s in range(N_DEV - 1):
            send_idx = jnp.mod(my - s, N_DEV)
            rdma = pltpu.make_async_remote_copy(
                src_ref=out_ref.at[pl.ds(send_idx * CHUNK, CHUNK), :],
                dst_ref=rs_buf.at[s],
                send_sem=rs_send_sems.at[s],
                recv_sem=rs_recv_sems.at[s],
                device_id=(right,),
                device_id_type=pltpu.DeviceIdType.MESH,
            )
            rdma.start()
            rdma.wait()
            recv_idx = jnp.mod(my - s - 1, N_DEV)
            off = recv_idx * CHUNK
            out_ref[pl.ds(off, CHUNK), :] = (
                out_ref[pl.ds(off, CHUNK), :] + rs_buf[s, :, :]
            )


        for s in range(N_DEV - 1):
            send_idx = jnp.mod(my + 1 - s, N_DEV)
            recv_idx = jnp.mod(my - s, N_DEV)
            rdma = pltpu.make_async_remote_copy(
                src_ref=out_ref.at[pl.ds(send_idx * CHUNK, CHUNK), :],
                dst_ref=out_ref.at[pl.ds(recv_idx * CHUNK, CHUNK), :],
                send_sem=ag_send_sems.at[s],
                recv_sem=ag_recv_sems.at[s],
                device_id=(right,),
                device_id_type=pltpu.DeviceIdType.MESH,
            )
            rdma.start()
            rdma.wait()

    return pl.pallas_call(
        body,
        out_shape=jax.ShapeDtypeStruct((ROWS, D_FF), jnp.float32),
        in_specs=[pl.BlockSpec(memory_space=pltpu.VMEM)] * 4,
        out_specs=pl.BlockSpec(memory_space=pltpu.VMEM),
        scratch_shapes=[
            pltpu.VMEM((N_DEV - 1, CHUNK, D_FF), jnp.float32),
            pltpu.SemaphoreType.DMA((N_DEV - 1,)),
            pltpu.SemaphoreType.DMA((N_DEV - 1,)),
            pltpu.SemaphoreType.DMA((N_DEV - 1,)),
            pltpu.SemaphoreType.DMA((N_DEV - 1,)),
        ],
        compiler_params=pltpu.CompilerParams(collective_id=0),
    )(x, router_W, route_idx, expert_W)


# baseline (device time: 54646 ns/iter reference)
import jax
import jax.numpy as jnp
from jax import lax
from jax.experimental import pallas as pl
from jax.experimental.pallas import tpu as pltpu

N_DEV = 8
ROWS = 512
D_MODEL = 256
D_FF = 512
N_EXP = 16
EXP_PER_DEV = N_EXP // N_DEV
CHUNK = ROWS // N_DEV


def kernel(x, router_W, route_idx, expert_W):
    def body(
        x_ref,
        rw_ref,
        idx_ref,
        ew_ref,
        out_ref,
        rs_buf,
        rs_send_sems,
        rs_recv_sems,
        ag_send_sems,
        ag_recv_sems,
    ):
        my = lax.axis_index("i")
        left = jnp.mod(my - 1, N_DEV)
        right = jnp.mod(my + 1, N_DEV)

        barrier_sem = pltpu.get_barrier_semaphore()
        for nbr in (left, right):
            pltpu.semaphore_signal(
                barrier_sem,
                inc=1,
                device_id=(nbr,),
                device_id_type=pltpu.DeviceIdType.MESH,
            )
        pltpu.semaphore_wait(barrier_sem, 2)

        x_v = x_ref[:, :]
        scores = jnp.dot(x_v, rw_ref[:, :], preferred_element_type=jnp.float32)
        s_max = jnp.max(scores, axis=-1, keepdims=True)
        p = jnp.exp(scores - s_max)
        p = p / jnp.sum(p, axis=-1, keepdims=True)

        idx0 = idx_ref[:, 0:1]
        idx1 = idx_ref[:, 1:2]
        e_iota = lax.broadcasted_iota(jnp.int32, (ROWS, N_EXP), 1)
        g0 = jnp.sum(
            jnp.where(idx0 == e_iota, p, 0.0), axis=-1, keepdims=True
        )
        g1 = jnp.sum(jnp.where(idx1 == e_iota, p, 0.0), axis=-1, keepdims=True)
        gs = g0 + g1
        w0 = g0 / gs
        w1 = g1 / gs

        e_base = my * EXP_PER_DEV
        acc = None
        for k in range(EXP_PER_DEV):
            e = e_base + k
            m = jnp.where(idx0 == e, w0, 0.0) + jnp.where(idx1 == e, w1, 0.0)
            xm = x_v * m
            c = jnp.dot(xm, ew_ref[k, :, :], preferred_element_type=jnp.float32)
            acc = c if acc is None else acc + c
        out_ref[:, :] = acc

        for s in range(N_DEV - 1):
            send_idx = jnp.mod(my - s, N_DEV)
            rdma = pltpu.make_async_remote_copy(
                src_ref=out_ref.at[pl.ds(send_idx * CHUNK, CHUNK), :],
                dst_ref=rs_buf.at[s],
                send_sem=rs_send_sems.at[s],
                recv_sem=rs_recv_sems.at[s],
                device_id=(right,),
                device_id_type=pltpu.DeviceIdType.MESH,
            )
            rdma.start()
            rdma.wait()
            recv_idx = jnp.mod(my - s - 1, N_DEV)
            off = recv_idx * CHUNK
            out_ref[pl.ds(off, CHUNK), :] = (
                out_ref[pl.ds(off, CHUNK), :] + rs_buf[s, :, :]
            )


        for s in range(N_DEV - 1):
            send_idx = jnp.mod(my + 1 - s, N_DEV)
            rdma = pltpu.make_async_remote_copy(
                src_ref=out_ref.at[pl.ds(send_idx * CHUNK, CHUNK), :],
                dst_ref=out_ref.at[pl.ds(send_idx * CHUNK, CHUNK), :],
                send_sem=ag_send_sems.at[s],
                recv_sem=ag_recv_sems.at[s],
                device_id=(right,),
                device_id_type=pltpu.DeviceIdType.MESH,
            )
            rdma.start()
            rdma.wait()

    return pl.pallas_call(
        body,
        out_shape=jax.ShapeDtypeStruct((ROWS, D_FF), jnp.float32),
        in_specs=[pl.BlockSpec(memory_space=pltpu.VMEM)] * 4,
        out_specs=pl.BlockSpec(memory_space=pltpu.VMEM),
        scratch_shapes=[
            pltpu.VMEM((N_DEV - 1, CHUNK, D_FF), jnp.float32),
            pltpu.SemaphoreType.DMA((N_DEV - 1,)),
            pltpu.SemaphoreType.DMA((N_DEV - 1,)),
            pltpu.SemaphoreType.DMA((N_DEV - 1,)),
            pltpu.SemaphoreType.DMA((N_DEV - 1,)),
        ],
        compiler_params=pltpu.CompilerParams(collective_id=0),
    )(x, router_W, route_idx, expert_W)


# device time: 39352 ns/iter; 1.3886x vs baseline; 1.3886x over previous
import jax
import jax.numpy as jnp
from jax import lax
from jax.experimental import pallas as pl
from jax.experimental.pallas import tpu as pltpu

N_DEV = 8
ROWS = 512
D_MODEL = 256
D_FF = 512
N_EXP = 16
EXP_PER_DEV = N_EXP // N_DEV


def kernel(x, router_W, route_idx, expert_W):
    def body(
        x_ref,
        rw_ref,
        idx_ref,
        ew_ref,
        out_ref,
        buf,
        send_sems,
        recv_sems,
    ):
        my = lax.axis_index("i")
        partners = (my ^ 1, my ^ 3, my ^ 4)

        barrier_sem = pltpu.get_barrier_semaphore()
        for prt in partners:
            pltpu.semaphore_signal(
                barrier_sem,
                inc=1,
                device_id=(prt,),
                device_id_type=pltpu.DeviceIdType.MESH,
            )
        pltpu.semaphore_wait(barrier_sem, 3)

        x_v = x_ref[:, :]
        scores = jnp.dot(x_v, rw_ref[:, :], preferred_element_type=jnp.float32)
        s_max = jnp.max(scores, axis=-1, keepdims=True)
        p = jnp.exp(scores - s_max)
        p = p / jnp.sum(p, axis=-1, keepdims=True)

        idx0 = idx_ref[:, 0:1]
        idx1 = idx_ref[:, 1:2]
        e_iota = lax.broadcasted_iota(jnp.int32, (ROWS, N_EXP), 1)
        g0 = jnp.sum(jnp.where(idx0 == e_iota, p, 0.0), axis=-1, keepdims=True)
        g1 = jnp.sum(jnp.where(idx1 == e_iota, p, 0.0), axis=-1, keepdims=True)
        gs = g0 + g1
        w0 = g0 / gs
        w1 = g1 / gs

        e_base = my * EXP_PER_DEV
        acc = None
        for k in range(EXP_PER_DEV):
            e = e_base + k
            m = jnp.where(idx0 == e, w0, 0.0) + jnp.where(idx1 == e, w1, 0.0)
            xm = x_v * m
            c = jnp.dot(xm, ew_ref[k, :, :], preferred_element_type=jnp.float32)
            acc = c if acc is None else acc + c
        out_ref[:, :] = acc

        b0 = my & 1
        b1 = (my >> 1) & 1
        b2 = (my >> 2) & 1
        steps = (
            (partners[0], 256, jnp.int32(0), b0),
            (partners[1], 128, b0 * 256, b1),
            (partners[2], 64, b0 * 256 + b1 * 128, b2),
        )

        for s, (prt, rows, base, bit) in enumerate(steps):
            keep_off = base + bit * rows
            send_off = base + (1 - bit) * rows
            rdma = pltpu.make_async_remote_copy(
                src_ref=out_ref.at[pl.ds(send_off, rows), :],
                dst_ref=buf.at[pl.ds(0, rows), :],
                send_sem=send_sems.at[s],
                recv_sem=recv_sems.at[s],
                device_id=(prt,),
                device_id_type=pltpu.DeviceIdType.MESH,
            )
            rdma.start()
            rdma.wait()
            out_ref[pl.ds(keep_off, rows), :] = (
                out_ref[pl.ds(keep_off, rows), :] + buf[pl.ds(0, rows), :]
            )

        for j, (prt, rows, base, bit) in enumerate(reversed(steps)):
            have_off = base + bit * rows
            rdma = pltpu.make_async_remote_copy(
                src_ref=out_ref.at[pl.ds(have_off, rows), :],
                dst_ref=out_ref.at[pl.ds(have_off, rows), :],
                send_sem=send_sems.at[3 + j],
                recv_sem=recv_sems.at[3 + j],
                device_id=(prt,),
                device_id_type=pltpu.DeviceIdType.MESH,
            )
            rdma.start()
            rdma.wait()

    return pl.pallas_call(
        body,
        out_shape=jax.ShapeDtypeStruct((ROWS, D_FF), jnp.float32),
        in_specs=[pl.BlockSpec(memory_space=pltpu.VMEM)] * 4,
        out_specs=pl.BlockSpec(memory_space=pltpu.VMEM),
        scratch_shapes=[
            pltpu.VMEM((ROWS // 2, D_FF), jnp.float32),
            pltpu.SemaphoreType.DMA((6,)),
            pltpu.SemaphoreType.DMA((6,)),
        ],
        compiler_params=pltpu.CompilerParams(collective_id=0),
    )(x, router_W, route_idx, expert_W)


# device time: 39349 ns/iter; 1.3888x vs baseline; 1.0001x over previous
import jax
import jax.numpy as jnp
from jax import lax
from jax.experimental import pallas as pl
from jax.experimental.pallas import tpu as pltpu

N_DEV = 8
ROWS = 512
D_MODEL = 256
D_FF = 512
N_EXP = 16
EXP_PER_DEV = N_EXP // N_DEV


def kernel(x, router_W, route_idx, expert_W):
    def body(
        x_ref,
        rw_ref,
        idx_ref,
        ew_ref,
        out_ref,
        buf,
        send_sems,
        recv_sems,
    ):
        my = lax.axis_index("i")
        partners = (my ^ 1, my ^ 3, my ^ 4)

        barrier_sem = pltpu.get_barrier_semaphore()
        for prt in partners:
            pltpu.semaphore_signal(
                barrier_sem,
                inc=1,
                device_id=(prt,),
                device_id_type=pltpu.DeviceIdType.MESH,
            )
        pltpu.semaphore_wait(barrier_sem, 3)

        x_v = x_ref[:, :]
        scores = jnp.dot(x_v, rw_ref[:, :], preferred_element_type=jnp.float32)
        s_max = jnp.max(scores, axis=-1, keepdims=True)
        p = jnp.exp(scores - s_max)
        p = p / jnp.sum(p, axis=-1, keepdims=True)

        idx0 = idx_ref[:, 0:1]
        idx1 = idx_ref[:, 1:2]
        e_iota = lax.broadcasted_iota(jnp.int32, (ROWS, N_EXP), 1)
        g0 = jnp.sum(jnp.where(idx0 == e_iota, p, 0.0), axis=-1, keepdims=True)
        g1 = jnp.sum(jnp.where(idx1 == e_iota, p, 0.0), axis=-1, keepdims=True)
        gs = g0 + g1
        w0 = g0 / gs
        w1 = g1 / gs

        e_base = my * EXP_PER_DEV
        acc = None
        for k in range(EXP_PER_DEV):
            e = e_base + k
            m = jnp.where(idx0 == e, w0, 0.0) + jnp.where(idx1 == e, w1, 0.0)
            xm = x_v * m
            c = jnp.dot(xm, ew_ref[k, :, :], preferred_element_type=jnp.float32)
            acc = c if acc is None else acc + c
        out_ref[:, :] = acc

        c0 = (my & 1) ^ ((my >> 1) & 1)
        c1 = (my >> 1) & 1
        c2 = (my >> 2) & 1
        steps = (
            (partners[0], 256, jnp.int32(0), c0),
            (partners[1], 128, c0 * 256, c1),
            (partners[2], 64, c0 * 256 + c1 * 128, c2),
        )

        for s, (prt, rows, base, bit) in enumerate(steps):
            keep_off = base + bit * rows
            send_off = base + (1 - bit) * rows
            rdma = pltpu.make_async_remote_copy(
                src_ref=out_ref.at[pl.ds(send_off, rows), :],
                dst_ref=buf.at[pl.ds(0, rows), :],
                send_sem=send_sems.at[s],
                recv_sem=recv_sems.at[s],
                device_id=(prt,),
                device_id_type=pltpu.DeviceIdType.MESH,
            )
            rdma.start()
            rdma.wait()
            out_ref[pl.ds(keep_off, rows), :] = (
                out_ref[pl.ds(keep_off, rows), :] + buf[pl.ds(0, rows), :]
            )

        for j, (prt, rows, base, bit) in enumerate(reversed(steps)):
            have_off = base + bit * rows
            rdma = pltpu.make_async_remote_copy(
                src_ref=out_ref.at[pl.ds(have_off, rows), :],
                dst_ref=out_ref.at[pl.ds(have_off, rows), :],
                send_sem=send_sems.at[3 + j],
                recv_sem=recv_sems.at[3 + j],
                device_id=(prt,),
                device_id_type=pltpu.DeviceIdType.MESH,
            )
            rdma.start()
            rdma.wait()

    return pl.pallas_call(
        body,
        out_shape=jax.ShapeDtypeStruct((ROWS, D_FF), jnp.float32),
        in_specs=[pl.BlockSpec(memory_space=pltpu.VMEM)] * 4,
        out_specs=pl.BlockSpec(memory_space=pltpu.VMEM),
        scratch_shapes=[
            pltpu.VMEM((ROWS // 2, D_FF), jnp.float32),
            pltpu.SemaphoreType.DMA((6,)),
            pltpu.SemaphoreType.DMA((6,)),
        ],
        compiler_params=pltpu.CompilerParams(collective_id=0),
    )(x, router_W, route_idx, expert_W)


# device time: 28733 ns/iter; 1.9019x vs baseline; 1.3695x over previous
import jax
import jax.numpy as jnp
from jax import lax
from jax.experimental import pallas as pl
from jax.experimental.pallas import tpu as pltpu

N_DEV = 8
ROWS = 512
D_MODEL = 256
D_FF = 512
N_EXP = 16
EXP_PER_DEV = N_EXP // N_DEV
HALF = ROWS // 4


def kernel(x, router_W, route_idx, expert_W):
    def body(
        x_ref,
        rw_ref,
        idx_ref,
        ew_ref,
        out_ref,
        buf,
        w_scr,
        send_sems,
        recv_sems,
    ):
        my = lax.axis_index("i")
        partners = (my ^ 1, my ^ 3, my ^ 4)

        barrier_sem = pltpu.get_barrier_semaphore()
        for prt in partners:
            pltpu.semaphore_signal(
                barrier_sem,
                inc=1,
                device_id=(prt,),
                device_id_type=pltpu.DeviceIdType.MESH,
            )

        x_v = x_ref[:, :]
        scores = jnp.dot(x_v, rw_ref[:, :], preferred_element_type=jnp.float32)
        s_max = jnp.max(scores, axis=-1, keepdims=True)
        p = jnp.exp(scores - s_max)
        p = p / jnp.sum(p, axis=-1, keepdims=True)

        idx0 = idx_ref[:, 0:1]
        idx1 = idx_ref[:, 1:2]
        e_iota = lax.broadcasted_iota(jnp.int32, (ROWS, N_EXP), 1)
        g0 = jnp.sum(jnp.where(idx0 == e_iota, p, 0.0), axis=-1, keepdims=True)
        g1 = jnp.sum(jnp.where(idx1 == e_iota, p, 0.0), axis=-1, keepdims=True)
        gs = g0 + g1
        w_scr[:, 0:1] = g0 / gs
        w_scr[:, 1:2] = g1 / gs

        e_base = my * EXP_PER_DEV

        def compute_rows(ro, nrows):
            xs = x_ref[pl.ds(ro, nrows), :]
            i0 = idx_ref[pl.ds(ro, nrows), 0:1]
            i1 = idx_ref[pl.ds(ro, nrows), 1:2]
            v0 = w_scr[pl.ds(ro, nrows), 0:1]
            v1 = w_scr[pl.ds(ro, nrows), 1:2]
            acc = None
            for k in range(EXP_PER_DEV):
                e = e_base + k
                m = jnp.where(i0 == e, v0, 0.0) + jnp.where(i1 == e, v1, 0.0)
                c = jnp.dot(
                    xs * m, ew_ref[k, :, :], preferred_element_type=jnp.float32
                )
                acc = c if acc is None else acc + c
            out_ref[pl.ds(ro, nrows), :] = acc

        sel_x = (my & 1) ^ ((my >> 1) & 1)
        sel_y = (my >> 1) & 1
        sel_z = (my >> 2) & 1

        def make_steps(sbase, gens):
            steps = []
            base = jnp.int32(sbase)
            rows = HALF
            for prt, sel in gens:
                steps.append((prt, rows, base, sel))
                base = base + sel * rows
                rows //= 2
            return steps

        streams = (
            make_steps(0, ((partners[0], sel_x), (partners[1], sel_y), (partners[2], sel_z))),
            make_steps(256, ((partners[2], sel_z), (partners[0], sel_x), (partners[1], sel_y))),
        )

        def rs_rdma(t, s):
            prt, rows, base, sel = streams[s][t]
            return pltpu.make_async_remote_copy(
                src_ref=out_ref.at[pl.ds(base + (1 - sel) * rows, rows), :],
                dst_ref=buf.at[pl.ds(s * HALF, rows), :],
                send_sem=send_sems.at[2 * t + s],
                recv_sem=recv_sems.at[2 * t + s],
                device_id=(prt,),
                device_id_type=pltpu.DeviceIdType.MESH,
            )

        def rs_add(t, s):
            _, rows, base, sel = streams[s][t]
            keep = base + sel * rows
            out_ref[pl.ds(keep, rows), :] = (
                out_ref[pl.ds(keep, rows), :]
                + buf[pl.ds(s * HALF, rows), :]
            )

        for s in (0, 1):
            _, rows, base, sel = streams[s][0]
            compute_rows(base + (1 - sel) * rows, HALF)

        barrier_sem = pltpu.get_barrier_semaphore()
        pltpu.semaphore_wait(barrier_sem, 3)

        step1 = [rs_rdma(0, s) for s in (0, 1)]
        for r in step1:
            r.start()

        for s in (0, 1):
            _, rows, base, sel = streams[s][0]
            compute_rows(base + sel * rows, HALF)

        for s in (0, 1):
            step1[s].wait()
            rs_add(0, s)

        for t in (1, 2):
            rdmas = [rs_rdma(t, s) for s in (0, 1)]
            for r in rdmas:
                r.start()
            for s in (0, 1):
                rdmas[s].wait()
                rs_add(t, s)

        for j in range(3):
            rdmas = []
            for s in (0, 1):
                prt, rows, base, sel = streams[s][2 - j]
                have = base + sel * rows
                rdmas.append(
                    pltpu.make_async_remote_copy(
                        src_ref=out_ref.at[pl.ds(have, rows), :],
                        dst_ref=out_ref.at[pl.ds(have, rows), :],
                        send_sem=send_sems.at[6 + 2 * j + s],
                        recv_sem=recv_sems.at[6 + 2 * j + s],
                        device_id=(prt,),
                        device_id_type=pltpu.DeviceIdType.MESH,
                    )
                )
            for r in rdmas:
                r.start()
            for r in rdmas:
                r.wait()

    return pl.pallas_call(
        body,
        out_shape=jax.ShapeDtypeStruct((ROWS, D_FF), jnp.float32),
        in_specs=[pl.BlockSpec(memory_space=pltpu.VMEM)] * 4,
        out_specs=pl.BlockSpec(memory_space=pltpu.VMEM),
        scratch_shapes=[
            pltpu.VMEM((2 * HALF, D_FF), jnp.float32),
            pltpu.VMEM((ROWS, 2), jnp.float32),
            pltpu.SemaphoreType.DMA((12,)),
            pltpu.SemaphoreType.DMA((12,)),
        ],
        compiler_params=pltpu.CompilerParams(collective_id=0),
    )(x, router_W, route_idx, expert_W)


# device time: 25491 ns/iter; 2.1437x vs baseline; 1.1272x over previous
import jax
import jax.numpy as jnp
from jax import lax
from jax.experimental import pallas as pl
from jax.experimental.pallas import tpu as pltpu

N_DEV = 8
ROWS = 512
D_MODEL = 256
D_FF = 512
N_EXP = 16
EXP_PER_DEV = N_EXP // N_DEV
HALF = ROWS // 4
CHALF = D_FF // 2
BUF_STEP_OFF = (0, HALF, HALF + HALF // 2)
BUF_ROWS_PER_STREAM = HALF + HALF // 2 + HALF // 4

SUBS = tuple((s, c) for c in (0, 1) for s in (0, 1))


def kernel(x, router_W, route_idx, expert_W):
    def body(
        x_ref,
        rw_ref,
        idx_ref,
        ew_ref,
        out_ref,
        buf,
        w_scr,
        send_sems,
        recv_sems,
    ):
        my = lax.axis_index("i")
        partners = (my ^ 1, my ^ 3, my ^ 4)

        barrier_sem = pltpu.get_barrier_semaphore()
        for prt in partners:
            pltpu.semaphore_signal(
                barrier_sem,
                inc=1,
                device_id=(prt,),
                device_id_type=pltpu.DeviceIdType.MESH,
            )

        x_v = x_ref[:, :]
        scores = jnp.dot(x_v, rw_ref[:, :], preferred_element_type=jnp.float32)
        s_max = jnp.max(scores, axis=-1, keepdims=True)
        p = jnp.exp(scores - s_max)
        p = p / jnp.sum(p, axis=-1, keepdims=True)

        idx0 = idx_ref[:, 0:1]
        idx1 = idx_ref[:, 1:2]
        e_iota = lax.broadcasted_iota(jnp.int32, (ROWS, N_EXP), 1)
        g0 = jnp.sum(jnp.where(idx0 == e_iota, p, 0.0), axis=-1, keepdims=True)
        g1 = jnp.sum(jnp.where(idx1 == e_iota, p, 0.0), axis=-1, keepdims=True)
        gs = g0 + g1
        w_scr[:, 0:1] = g0 / gs
        w_scr[:, 1:2] = g1 / gs

        e_base = my * EXP_PER_DEV

        def compute_rows(ro, nrows):
            xs = x_ref[pl.ds(ro, nrows), :]
            i0 = idx_ref[pl.ds(ro, nrows), 0:1]
            i1 = idx_ref[pl.ds(ro, nrows), 1:2]
            v0 = w_scr[pl.ds(ro, nrows), 0:1]
            v1 = w_scr[pl.ds(ro, nrows), 1:2]
            acc = None
            for k in range(EXP_PER_DEV):
                e = e_base + k
                m = jnp.where(i0 == e, v0, 0.0) + jnp.where(i1 == e, v1, 0.0)
                c = jnp.dot(
                    xs * m, ew_ref[k, :, :], preferred_element_type=jnp.float32
                )
                acc = c if acc is None else acc + c
            out_ref[pl.ds(ro, nrows), :] = acc

        sel_x = (my & 1) ^ ((my >> 1) & 1)
        sel_y = (my >> 1) & 1
        sel_z = (my >> 2) & 1

        def make_steps(sbase, gens):
            steps = []
            base = jnp.int32(sbase)
            rows = HALF
            for prt, sel in gens:
                steps.append((prt, rows, base, sel))
                base = base + sel * rows
                rows //= 2
            return steps

        streams = (
            make_steps(0, ((partners[0], sel_x), (partners[1], sel_y), (partners[2], sel_z))),
            make_steps(256, ((partners[2], sel_z), (partners[0], sel_x), (partners[1], sel_y))),
        )

        def rs_rdma(t, s, c):
            prt, rows, base, sel = streams[s][t]
            cs = pl.ds(c * CHALF, CHALF)
            return pltpu.make_async_remote_copy(
                src_ref=out_ref.at[pl.ds(base + (1 - sel) * rows, rows), cs],
                dst_ref=buf.at[
                    pl.ds(s * BUF_ROWS_PER_STREAM + BUF_STEP_OFF[t], rows), cs
                ],
                send_sem=send_sems.at[4 * t + 2 * s + c],
                recv_sem=recv_sems.at[4 * t + 2 * s + c],
                device_id=(prt,),
                device_id_type=pltpu.DeviceIdType.MESH,
            )

        def rs_add(t, s, c):
            _, rows, base, sel = streams[s][t]
            keep = base + sel * rows
            cs = pl.ds(c * CHALF, CHALF)
            bo = s * BUF_ROWS_PER_STREAM + BUF_STEP_OFF[t]
            out_ref[pl.ds(keep, rows), cs] = (
                out_ref[pl.ds(keep, rows), cs] + buf[pl.ds(bo, rows), cs]
            )

        def ag_rdma(j, s, c):
            prt, rows, base, sel = streams[s][2 - j]
            have = base + sel * rows
            cs = pl.ds(c * CHALF, CHALF)
            return pltpu.make_async_remote_copy(
                src_ref=out_ref.at[pl.ds(have, rows), cs],
                dst_ref=out_ref.at[pl.ds(have, rows), cs],
                send_sem=send_sems.at[12 + 4 * j + 2 * s + c],
                recv_sem=recv_sems.at[12 + 4 * j + 2 * s + c],
                device_id=(prt,),
                device_id_type=pltpu.DeviceIdType.MESH,
            )

        for s in (0, 1):
            _, rows, base, sel = streams[s][0]
            compute_rows(base + (1 - sel) * rows, HALF)

        pltpu.semaphore_wait(barrier_sem, 3)

        cur = {sc: rs_rdma(0, *sc) for sc in SUBS}
        for r in cur.values():
            r.start()

        for s in (0, 1):
            _, rows, base, sel = streams[s][0]
            compute_rows(base + sel * rows, HALF)

        ag = {}
        for t in range(3):
            nxt = {}
            for sc in SUBS:
                cur[sc].wait()
                rs_add(t, *sc)
                if t < 2:
                    nxt[sc] = rs_rdma(t + 1, *sc)
                    nxt[sc].start()
                else:
                    ag[sc] = ag_rdma(0, *sc)
                    ag[sc].start()
            cur = nxt

        for j in range(3):
            nxt = {}
            for sc in SUBS:
                ag[sc].wait()
                if j < 2:
                    nxt[sc] = ag_rdma(j + 1, *sc)
                    nxt[sc].start()
            ag = nxt

    return pl.pallas_call(
        body,
        out_shape=jax.ShapeDtypeStruct((ROWS, D_FF), jnp.float32),
        in_specs=[pl.BlockSpec(memory_space=pltpu.VMEM)] * 4,
        out_specs=pl.BlockSpec(memory_space=pltpu.VMEM),
        scratch_shapes=[
            pltpu.VMEM((2 * BUF_ROWS_PER_STREAM, D_FF), jnp.float32),
            pltpu.VMEM((ROWS, 2), jnp.float32),
            pltpu.SemaphoreType.DMA((24,)),
            pltpu.SemaphoreType.DMA((24,)),
        ],
        compiler_params=pltpu.CompilerParams(collective_id=0),
    )(x, router_W, route_idx, expert_W)


# device time: 25175 ns/iter; 2.1706x vs baseline; 1.0126x over previous
import os

import jax
import jax.numpy as jnp
from jax import lax
from jax.experimental import pallas as pl
from jax.experimental.pallas import tpu as pltpu

_KMODE = os.environ.get("KMODE", "full")

N_DEV = 8
ROWS = 512
D_MODEL = 256
D_FF = 512
N_EXP = 16
EXP_PER_DEV = N_EXP // N_DEV
HALF = ROWS // 4
CHALF = D_FF // 2
BUF_STEP_OFF = (0, HALF, HALF + HALF // 2)
BUF_ROWS_PER_STREAM = HALF + HALF // 2 + HALF // 4

SUBS = tuple((s, c) for c in (0, 1) for s in (0, 1))


def kernel(x, router_W, route_idx, expert_W):
    def body(
        x_ref,
        rw_ref,
        idx_ref,
        ew_ref,
        out_ref,
        buf,
        w_scr,
        send_sems,
        recv_sems,
    ):
        my = lax.axis_index("i")
        partners = (my ^ 1, my ^ 3, my ^ 4)

        barrier_sem = pltpu.get_barrier_semaphore()
        for prt in partners:
            pltpu.semaphore_signal(
                barrier_sem,
                inc=1,
                device_id=(prt,),
                device_id_type=pltpu.DeviceIdType.MESH,
            )

        x_v = x_ref[:, :]
        scores = jnp.dot(x_v, rw_ref[:, :], preferred_element_type=jnp.float32)
        s_max = jnp.max(scores, axis=-1, keepdims=True)
        p = jnp.exp(scores - s_max)
        p = p / jnp.sum(p, axis=-1, keepdims=True)

        idx0 = idx_ref[:, 0:1]
        idx1 = idx_ref[:, 1:2]
        e_iota = lax.broadcasted_iota(jnp.int32, (ROWS, N_EXP), 1)
        g0 = jnp.sum(jnp.where(idx0 == e_iota, p, 0.0), axis=-1, keepdims=True)
        g1 = jnp.sum(jnp.where(idx1 == e_iota, p, 0.0), axis=-1, keepdims=True)
        gs = g0 + g1
        w_scr[:, 0:1] = g0 / gs
        w_scr[:, 1:2] = g1 / gs

        e_base = my * EXP_PER_DEV

        def compute_rows(ro, nrows):
            xs = x_ref[pl.ds(ro, nrows), :]
            i0 = idx_ref[pl.ds(ro, nrows), 0:1]
            i1 = idx_ref[pl.ds(ro, nrows), 1:2]
            v0 = w_scr[pl.ds(ro, nrows), 0:1]
            v1 = w_scr[pl.ds(ro, nrows), 1:2]
            acc = None
            n_exp = 1 if _KMODE == "nocompute" else EXP_PER_DEV
            for k in range(n_exp):
                e = e_base + k
                m = jnp.where(i0 == e, v0, 0.0) + jnp.where(i1 == e, v1, 0.0)
                c = jnp.dot(
                    xs * m, ew_ref[k, :, :], preferred_element_type=jnp.float32
                )
                acc = c if acc is None else acc + c
            out_ref[pl.ds(ro, nrows), :] = acc

        sel_x = (my & 1) ^ ((my >> 1) & 1)
        sel_y = (my >> 1) & 1
        sel_z = (my >> 2) & 1

        def make_steps(sbase, gens):
            steps = []
            base = jnp.int32(sbase)
            rows = HALF
            for prt, sel in gens:
                steps.append((prt, rows, base, sel))
                base = base + sel * rows
                rows //= 2
            return steps

        streams = (
            make_steps(0, ((partners[0], sel_x), (partners[1], sel_y), (partners[2], sel_z))),
            make_steps(256, ((partners[2], sel_z), (partners[0], sel_x), (partners[1], sel_y))),
        )

        def rs_rdma(t, s, c):
            prt, rows, base, sel = streams[s][t]
            cs = pl.ds(c * CHALF, CHALF)
            return pltpu.make_async_remote_copy(
                src_ref=out_ref.at[pl.ds(base + (1 - sel) * rows, rows), cs],
                dst_ref=buf.at[
                    pl.ds(s * BUF_ROWS_PER_STREAM + BUF_STEP_OFF[t], rows), cs
                ],
                send_sem=send_sems.at[4 * t + 2 * s + c],
                recv_sem=recv_sems.at[4 * t + 2 * s + c],
                device_id=(prt,),
                device_id_type=pltpu.DeviceIdType.MESH,
            )

        def rs_add(t, s, c):
            _, rows, base, sel = streams[s][t]
            keep = base + sel * rows
            cs = pl.ds(c * CHALF, CHALF)
            bo = s * BUF_ROWS_PER_STREAM + BUF_STEP_OFF[t]
            out_ref[pl.ds(keep, rows), cs] = (
                out_ref[pl.ds(keep, rows), cs] + buf[pl.ds(bo, rows), cs]
            )

        def ag_rdma(j, s, c):
            prt, rows, base, sel = streams[s][2 - j]
            have = base + sel * rows
            cs = pl.ds(c * CHALF, CHALF)
            return pltpu.make_async_remote_copy(
                src_ref=out_ref.at[pl.ds(have, rows), cs],
                dst_ref=out_ref.at[pl.ds(have, rows), cs],
                send_sem=send_sems.at[12 + 4 * j + 2 * s + c],
                recv_sem=recv_sems.at[12 + 4 * j + 2 * s + c],
                device_id=(prt,),
                device_id_type=pltpu.DeviceIdType.MESH,
            )

        for s in (0, 1):
            _, rows, base, sel = streams[s][0]
            compute_rows(base + (1 - sel) * rows, HALF)

        pltpu.semaphore_wait(barrier_sem, 3)

        if _KMODE == "nocomm":
            for s in (0, 1):
                _, rows, base, sel = streams[s][0]
                compute_rows(base + sel * rows, HALF)
            return

        cur = {sc: rs_rdma(0, *sc) for sc in SUBS}
        for r in cur.values():
            r.start()

        for s in (0, 1):
            _, rows, base, sel = streams[s][0]
            compute_rows(base + sel * rows, HALF)

        ag = {}
        for t in range(3):
            nxt = {}
            for sc in SUBS:
                cur[sc].wait()
                rs_add(t, *sc)
                if t < 2:
                    nxt[sc] = rs_rdma(t + 1, *sc)
                    nxt[sc].start()
                else:
                    ag[sc] = ag_rdma(0, *sc)
                    ag[sc].start()
            cur = nxt

        for j in range(3):
            nxt = {}
            for sc in SUBS:
                ag[sc].wait()
                if j < 2:
                    nxt[sc] = ag_rdma(j + 1, *sc)
                    nxt[sc].start()
            ag = nxt

    return pl.pallas_call(
        body,
        out_shape=jax.ShapeDtypeStruct((ROWS, D_FF), jnp.float32),
        in_specs=[pl.BlockSpec(memory_space=pltpu.VMEM)] * 4,
        out_specs=pl.BlockSpec(memory_space=pltpu.VMEM),
        scratch_shapes=[
            pltpu.VMEM((2 * BUF_ROWS_PER_STREAM, D_FF), jnp.float32),
            pltpu.VMEM((ROWS, 2), jnp.float32),
            pltpu.SemaphoreType.DMA((24,)),
            pltpu.SemaphoreType.DMA((24,)),
        ],
        compiler_params=pltpu.CompilerParams(collective_id=0),
    )(x, router_W, route_idx, expert_W)


# device time: 20648 ns/iter; 2.6466x vs baseline; 1.2192x over previous
import os

import jax
import jax.numpy as jnp
from jax import lax
from jax.experimental import pallas as pl
from jax.experimental.pallas import tpu as pltpu

_KMODE = os.environ.get("KMODE", "full")

N_DEV = 8
ROWS = 512
D_MODEL = 256
D_FF = 512
N_EXP = 16
EXP_PER_DEV = N_EXP // N_DEV
SROWS = ROWS // 2
CHALF = D_FF // 2

SUBS = ((0, 0), (1, 0), (0, 1), (1, 1))


def kernel(x, router_W, route_idx, expert_W):
    def body(
        x_ref,
        rw_ref,
        idx_ref,
        ew_ref,
        out_ref,
        pbuf,
        rbuf,
        send_sems,
        recv_sems,
    ):
        my = lax.axis_index("i")
        partners = (my ^ 1, my ^ 3, my ^ 4)
        orders = ((0, 1, 2), (2, 0, 1))

        barrier_sem = pltpu.get_barrier_semaphore()
        for prt in partners:
            pltpu.semaphore_signal(
                barrier_sem,
                inc=1,
                device_id=(prt,),
                device_id_type=pltpu.DeviceIdType.MESH,
            )

        x_v = x_ref[:, :]
        scores = jnp.dot(x_v, rw_ref[:, :], preferred_element_type=jnp.float32)
        s_max = jnp.max(scores, axis=-1, keepdims=True)
        p = jnp.exp(scores - s_max)
        p = p / jnp.sum(p, axis=-1, keepdims=True)

        idx0 = idx_ref[:, 0:1]
        idx1 = idx_ref[:, 1:2]
        e_iota = lax.broadcasted_iota(jnp.int32, (ROWS, N_EXP), 1)
        g0 = jnp.sum(jnp.where(idx0 == e_iota, p, 0.0), axis=-1, keepdims=True)
        g1 = jnp.sum(jnp.where(idx1 == e_iota, p, 0.0), axis=-1, keepdims=True)
        gs = g0 + g1
        w0 = g0 / gs
        w1 = g1 / gs

        e_base = my * EXP_PER_DEV
        ew_bf = [
            ew_ref[k, :, :].astype(jnp.bfloat16) for k in range(EXP_PER_DEV)
        ]

        def compute_stream(s):
            ro = s * SROWS
            xs = x_v[ro : ro + SROWS, :]
            i0 = idx0[ro : ro + SROWS, :]
            i1 = idx1[ro : ro + SROWS, :]
            v0 = w0[ro : ro + SROWS, :]
            v1 = w1[ro : ro + SROWS, :]
            acc = None
            for k in range(EXP_PER_DEV):
                e = e_base + k
                m = jnp.where(i0 == e, v0, 0.0) + jnp.where(i1 == e, v1, 0.0)
                xm = (xs * m).astype(jnp.bfloat16)
                c = jnp.dot(xm, ew_bf[k], preferred_element_type=jnp.float32)
                acc = c if acc is None else acc + c
            pbuf[ro : ro + SROWS, :] = acc.astype(jnp.bfloat16)

        def xc_rdma(t, s, c):
            cs = pl.ds(c * CHALF, CHALF)
            return pltpu.make_async_remote_copy(
                src_ref=pbuf.at[pl.ds(s * SROWS, SROWS), cs],
                dst_ref=rbuf.at[pl.ds((s * 3 + t) * SROWS, SROWS), cs],
                send_sem=send_sems.at[4 * t + 2 * s + c],
                recv_sem=recv_sems.at[4 * t + 2 * s + c],
                device_id=(partners[orders[s][t]],),
                device_id_type=pltpu.DeviceIdType.MESH,
            )

        def xc_add(t, s, c):
            cs = pl.ds(c * CHALF, CHALF)
            ro = s * SROWS
            reg = (s * 3 + t) * SROWS
            pbuf[pl.ds(ro, SROWS), cs] = (
                pbuf[pl.ds(ro, SROWS), cs] + rbuf[pl.ds(reg, SROWS), cs]
            )

        compute_stream(0)
        pltpu.semaphore_wait(barrier_sem, 3)

        if _KMODE == "nocomm":
            compute_stream(1)
            out_ref[:, :] = pbuf[:, :].astype(jnp.float32)
            return

        cur = {}
        for c in (0, 1):
            cur[(0, c)] = xc_rdma(0, 0, c)
            cur[(0, c)].start()
        compute_stream(1)
        for c in (0, 1):
            cur[(1, c)] = xc_rdma(0, 1, c)
            cur[(1, c)].start()

        for t in range(3):
            nxt = {}
            for sc in SUBS:
                cur[sc].wait()
                xc_add(t, *sc)
                if t < 2:
                    nxt[sc] = xc_rdma(t + 1, *sc)
                    nxt[sc].start()
            cur = nxt

        out_ref[:, :] = pbuf[:, :].astype(jnp.float32)

    return pl.pallas_call(
        body,
        out_shape=jax.ShapeDtypeStruct((ROWS, D_FF), jnp.float32),
        in_specs=[pl.BlockSpec(memory_space=pltpu.VMEM)] * 4,
        out_specs=pl.BlockSpec(memory_space=pltpu.VMEM),
        scratch_shapes=[
            pltpu.VMEM((ROWS, D_FF), jnp.bfloat16),
            pltpu.VMEM((2 * 3 * SROWS, D_FF), jnp.bfloat16),
            pltpu.SemaphoreType.DMA((12,)),
            pltpu.SemaphoreType.DMA((12,)),
        ],
        compiler_params=pltpu.CompilerParams(collective_id=0),
    )(x, router_W, route_idx, expert_W)


# device time: 18570 ns/iter; 2.9427x vs baseline; 1.1119x over previous
import os

import jax
import jax.numpy as jnp
from jax import lax
from jax.experimental import pallas as pl
from jax.experimental.pallas import tpu as pltpu

_KMODE = os.environ.get("KMODE", "full")

N_DEV = 8
ROWS = 512
D_MODEL = 256
D_FF = 512
N_EXP = 16
EXP_PER_DEV = N_EXP // N_DEV
CHALF = D_FF // 2

STREAMS = ((0, 160), (160, 176), (336, 176))
N_STREAMS = 3
RBUF_BASE = []
_off = 0
for _b, _r in STREAMS:
    RBUF_BASE.append(tuple(_off + t * _r for t in range(3)))
    _off += 3 * _r
RBUF_ROWS = _off

SUBS = tuple((s, c) for c in (0, 1) for s in range(N_STREAMS))


def kernel(x, router_W, route_idx, expert_W):
    def body(
        x_ref,
        rw_ref,
        idx_ref,
        ew_ref,
        out_ref,
        pbuf,
        rbuf,
        send_sems,
        recv_sems,
    ):
        my = lax.axis_index("i")
        partners = (my ^ 1, my ^ 3, my ^ 4)
        orders = ((0, 1, 2), (1, 2, 0), (2, 0, 1))

        if _KMODE == "min":
            out_ref[:, :] = jnp.zeros((ROWS, D_FF), jnp.float32)
            return

        if _KMODE != "nobarrier":
            barrier_sem = pltpu.get_barrier_semaphore()
            for prt in partners:
                pltpu.semaphore_signal(
                    barrier_sem,
                    inc=1,
                    device_id=(prt,),
                    device_id_type=pltpu.DeviceIdType.MESH,
                )

        x_v = x_ref[:, :]
        scores = jnp.dot(x_v, rw_ref[:, :], preferred_element_type=jnp.float32)
        s_max = jnp.max(scores, axis=-1, keepdims=True)
        p = jnp.exp(scores - s_max)
        p = p / jnp.sum(p, axis=-1, keepdims=True)

        idx0 = idx_ref[:, 0:1]
        idx1 = idx_ref[:, 1:2]
        e_iota = lax.broadcasted_iota(jnp.int32, (ROWS, N_EXP), 1)
        g0 = jnp.sum(jnp.where(idx0 == e_iota, p, 0.0), axis=-1, keepdims=True)
        g1 = jnp.sum(jnp.where(idx1 == e_iota, p, 0.0), axis=-1, keepdims=True)
        gs = g0 + g1
        w0 = g0 / gs
        w1 = g1 / gs

        e_base = my * EXP_PER_DEV
        ew_bf = [
            ew_ref[k, :, :].astype(jnp.bfloat16) for k in range(EXP_PER_DEV)
        ]

        def compute_stream(s):
            ro, nrows = STREAMS[s]
            xs = x_v[ro : ro + nrows, :]
            i0 = idx0[ro : ro + nrows, :]
            i1 = idx1[ro : ro + nrows, :]
            v0 = w0[ro : ro + nrows, :]
            v1 = w1[ro : ro + nrows, :]
            acc = None
            for k in range(EXP_PER_DEV):
                e = e_base + k
                m = jnp.where(i0 == e, v0, 0.0) + jnp.where(i1 == e, v1, 0.0)
                xm = (xs * m).astype(jnp.bfloat16)
                c = jnp.dot(xm, ew_bf[k], preferred_element_type=jnp.float32)
                acc = c if acc is None else acc + c
            pbuf[ro : ro + nrows, :] = acc.astype(jnp.bfloat16)

        def sem_idx(t, s, c):
            return 6 * t + 2 * s + c

        def xc_rdma(t, s, c):
            ro, nrows = STREAMS[s]
            cs = pl.ds(c * CHALF, CHALF)
            return pltpu.make_async_remote_copy(
                src_ref=pbuf.at[pl.ds(ro, nrows), cs],
                dst_ref=rbuf.at[pl.ds(RBUF_BASE[s][t], nrows), cs],
                send_sem=send_sems.at[sem_idx(t, s, c)],
                recv_sem=recv_sems.at[sem_idx(t, s, c)],
                device_id=(partners[orders[s][t]],),
                device_id_type=pltpu.DeviceIdType.MESH,
            )

        def xc_add(t, s, c):
            ro, nrows = STREAMS[s]
            cs = pl.ds(c * CHALF, CHALF)
            pbuf[pl.ds(ro, nrows), cs] = (
                pbuf[pl.ds(ro, nrows), cs]
                + rbuf[pl.ds(RBUF_BASE[s][t], nrows), cs]
            )

        compute_stream(0)
        if _KMODE != "nobarrier":
            pltpu.semaphore_wait(barrier_sem, 3)

        if _KMODE in ("nocomm", "nobarrier"):
            compute_stream(1)
            compute_stream(2)
            out_ref[:, :] = pbuf[:, :].astype(jnp.float32)
            return

        cur = {}
        for s in range(N_STREAMS):
            if s > 0:
                compute_stream(s)
            for c in (0, 1):
                cur[(s, c)] = xc_rdma(0, s, c)
                cur[(s, c)].start()

        for t in range(3):
            nxt = {}
            for sc in SUBS:
                cur[sc].wait()
                xc_add(t, *sc)
                if t < 2:
                    nxt[sc] = xc_rdma(t + 1, *sc)
                    nxt[sc].start()
            cur = nxt

        out_ref[:, :] = pbuf[:, :].astype(jnp.float32)

    return pl.pallas_call(
        body,
        out_shape=jax.ShapeDtypeStruct((ROWS, D_FF), jnp.float32),
        in_specs=[pl.BlockSpec(memory_space=pltpu.VMEM)] * 4,
        out_specs=pl.BlockSpec(memory_space=pltpu.VMEM),
        scratch_shapes=[
            pltpu.VMEM((ROWS, D_FF), jnp.bfloat16),
            pltpu.VMEM((RBUF_ROWS, D_FF), jnp.bfloat16),
            pltpu.SemaphoreType.DMA((18,)),
            pltpu.SemaphoreType.DMA((18,)),
        ],
        compiler_params=(
            pltpu.CompilerParams()
            if _KMODE in ("min", "nobarrier")
            else pltpu.CompilerParams(collective_id=0)
        ),
    )(x, router_W, route_idx, expert_W)


# device time: 17815 ns/iter; 3.0674x vs baseline; 1.0424x over previous
import os

import jax
import jax.numpy as jnp
from jax import lax
from jax.experimental import pallas as pl
from jax.experimental.pallas import tpu as pltpu

_KMODE = os.environ.get("KMODE", "full")

N_DEV = 8
ROWS = 512
D_MODEL = 256
D_FF = 512
N_EXP = 16
EXP_PER_DEV = N_EXP // N_DEV
N_CSUB = 4
CSUB = D_FF // N_CSUB

STREAMS = ((0, 160), (160, 176), (336, 176))
N_STREAMS = 3
RBUF_BASE = []
_off = 0
for _b, _r in STREAMS:
    RBUF_BASE.append(tuple(_off + t * _r for t in range(3)))
    _off += 3 * _r
RBUF_ROWS = _off

SUBS = tuple((s, c) for c in range(N_CSUB) for s in range(N_STREAMS))


def kernel(x, router_W, route_idx, expert_W):
    def body(
        x_ref,
        rw_ref,
        idx_ref,
        ew_ref,
        out_ref,
        pbuf,
        rbuf,
        send_sems,
        recv_sems,
    ):
        my = lax.axis_index("i")
        partners = (my ^ 1, my ^ 3, my ^ 4)
        orders = ((0, 1, 2), (1, 2, 0), (2, 0, 1))

        if _KMODE == "min":
            out_ref[:, :] = jnp.zeros((ROWS, D_FF), jnp.float32)
            return

        if _KMODE != "nobarrier":
            barrier_sem = pltpu.get_barrier_semaphore()
            for prt in partners:
                pltpu.semaphore_signal(
                    barrier_sem,
                    inc=1,
                    device_id=(prt,),
                    device_id_type=pltpu.DeviceIdType.MESH,
                )

        x_v = x_ref[:, :]
        scores = jnp.dot(x_v, rw_ref[:, :], preferred_element_type=jnp.float32)
        s_max = jnp.max(scores, axis=-1, keepdims=True)
        p = jnp.exp(scores - s_max)
        p = p / jnp.sum(p, axis=-1, keepdims=True)

        idx0 = idx_ref[:, 0:1]
        idx1 = idx_ref[:, 1:2]
        e_iota = lax.broadcasted_iota(jnp.int32, (ROWS, N_EXP), 1)
        g0 = jnp.sum(jnp.where(idx0 == e_iota, p, 0.0), axis=-1, keepdims=True)
        g1 = jnp.sum(jnp.where(idx1 == e_iota, p, 0.0), axis=-1, keepdims=True)
        gs = g0 + g1
        w0 = g0 / gs
        w1 = g1 / gs

        e_base = my * EXP_PER_DEV
        ew_bf = [
            ew_ref[k, :, :].astype(jnp.bfloat16) for k in range(EXP_PER_DEV)
        ]

        def compute_stream(s):
            ro, nrows = STREAMS[s]
            xs = x_v[ro : ro + nrows, :]
            i0 = idx0[ro : ro + nrows, :]
            i1 = idx1[ro : ro + nrows, :]
            v0 = w0[ro : ro + nrows, :]
            v1 = w1[ro : ro + nrows, :]
            acc = None
            for k in range(EXP_PER_DEV):
                e = e_base + k
                m = jnp.where(i0 == e, v0, 0.0) + jnp.where(i1 == e, v1, 0.0)
                xm = (xs * m).astype(jnp.bfloat16)
                c = jnp.dot(xm, ew_bf[k], preferred_element_type=jnp.float32)
                acc = c if acc is None else acc + c
            pbuf[ro : ro + nrows, :] = acc.astype(jnp.bfloat16)

        def sem_idx(t, s, c):
            return N_CSUB * (N_STREAMS * t + s) + c

        def xc_rdma(t, s, c):
            ro, nrows = STREAMS[s]
            cs = pl.ds(c * CSUB, CSUB)
            return pltpu.make_async_remote_copy(
                src_ref=pbuf.at[pl.ds(ro, nrows), cs],
                dst_ref=rbuf.at[pl.ds(RBUF_BASE[s][t], nrows), cs],
                send_sem=send_sems.at[sem_idx(t, s, c)],
                recv_sem=recv_sems.at[sem_idx(t, s, c)],
                device_id=(partners[orders[s][t]],),
                device_id_type=pltpu.DeviceIdType.MESH,
            )

        def xc_add(t, s, c):
            ro, nrows = STREAMS[s]
            cs = pl.ds(c * CSUB, CSUB)
            total = (
                pbuf[pl.ds(ro, nrows), cs]
                + rbuf[pl.ds(RBUF_BASE[s][t], nrows), cs]
            )
            if t < 2:
                pbuf[pl.ds(ro, nrows), cs] = total
            else:
                out_ref[pl.ds(ro, nrows), cs] = total.astype(jnp.float32)

        compute_stream(0)
        if _KMODE != "nobarrier":
            pltpu.semaphore_wait(barrier_sem, 3)

        if _KMODE in ("nocomm", "nobarrier"):
            compute_stream(1)
            compute_stream(2)
            out_ref[:, :] = pbuf[:, :].astype(jnp.float32)
            return

        cur = {}
        for s in range(N_STREAMS):
            if s > 0:
                compute_stream(s)
            for c in range(N_CSUB):
                cur[(s, c)] = xc_rdma(0, s, c)
                cur[(s, c)].start()

        for t in range(3):
            nxt = {}
            for sc in SUBS:
                cur[sc].wait()
                xc_add(t, *sc)
                if t < 2:
                    nxt[sc] = xc_rdma(t + 1, *sc)
                    nxt[sc].start()
            cur = nxt

    return pl.pallas_call(
        body,
        out_shape=jax.ShapeDtypeStruct((ROWS, D_FF), jnp.float32),
        in_specs=[pl.BlockSpec(memory_space=pltpu.VMEM)] * 4,
        out_specs=pl.BlockSpec(memory_space=pltpu.VMEM),
        scratch_shapes=[
            pltpu.VMEM((ROWS, D_FF), jnp.bfloat16),
            pltpu.VMEM((RBUF_ROWS, D_FF), jnp.bfloat16),
            pltpu.SemaphoreType.DMA((36,)),
            pltpu.SemaphoreType.DMA((36,)),
        ],
        compiler_params=(
            pltpu.CompilerParams()
            if _KMODE in ("min", "nobarrier")
            else pltpu.CompilerParams(collective_id=0)
        ),
    )(x, router_W, route_idx, expert_W)
